# baseline (speedup 1.0000x reference)
import functools

import jax
import jax.numpy as jnp
from jax.experimental import pallas as pl
from jax.experimental.pallas import tpu as pltpu
from jax.experimental.pallas import tpu_sc as plsc

_EPS = 1e-5
_WIN = 128
_NWORK = 32
_BLK = 200
_BLK3 = 1000


def _sc_gather(table, idx_pad, epad):
    a = table.shape[1]
    mesh = plsc.VectorSubcoreMesh(core_axis_name="core", subcore_axis_name="subcore")

    steps = epad // _WIN // _NWORK

    @functools.partial(
        pl.kernel,
        out_type=jax.ShapeDtypeStruct((epad, a), table.dtype),
        mesh=mesh,
    )
    def gather_kernel(x_hbm, i_hbm, o_hbm):
        def body(i_vmem, o_vmem):
            pltpu.sync_copy(x_hbm.at[i_vmem.at[0]], o_vmem)

        pltpu.emit_pipeline(
            body,
            grid=(_NWORK, steps),
            in_specs=[pl.BlockSpec((1, _WIN), lambda i, j: (0, i * steps + j))],
            out_specs=[pl.BlockSpec((_WIN, a), lambda i, j: (i * steps + j, 0))],
            core_axis_name=("core", "subcore"),
            dimension_semantics=(pltpu.PARALLEL, pltpu.ARBITRARY),
        )(i_hbm, o_hbm)

    return gather_kernel(table, idx_pad)


def _p1_kernel(x, w, b2d):
    n, a = x.shape
    c = w.shape[1]
    blk = _BLK3

    def body(x_ref, w_ref, b_ref, o_ref, xb_ref):
        w1 = w_ref[0:a, :]
        o_ref[...] = (
            jnp.dot(x_ref[...], w1, preferred_element_type=jnp.float32)
            + b_ref[...]
        )
        xb_ref[...] = x_ref[...].astype(jnp.bfloat16)

    return pl.pallas_call(
        body,
        grid=(n // blk,),
        in_specs=[
            pl.BlockSpec((blk, a), lambda i: (i, 0)),
            pl.BlockSpec(w.shape, lambda i: (0, 0)),
            pl.BlockSpec(b2d.shape, lambda i: (0, 0)),
        ],
        out_specs=[
            pl.BlockSpec((blk, c), lambda i: (i, 0)),
            pl.BlockSpec((blk, a), lambda i: (i, 0)),
        ],
        out_shape=[
            jax.ShapeDtypeStruct((n, c), jnp.float32),
            jax.ShapeDtypeStruct((n, a), jnp.bfloat16),
        ],
    )(x, w, b2d)


def _edge_rows(g_ref, nb_ref, p1_ref, w_ref, a, m, blk):
    w2 = w_ref[a : 2 * a, :]
    w3 = w_ref[2 * a :, :]
    g = jnp.dot(g_ref[...], w2, preferred_element_type=jnp.float32)
    g = g + jnp.dot(nb_ref[...], w3, preferred_element_type=jnp.float32)
    g3 = g.reshape(blk, m, 2 * a) + p1_ref[...][:, None, :]
    return g3


def _stats_pass(g_rows, nbr2, p1, w, n, m, a):
    blk = _BLK
    eblk = blk * m
    grid = n // blk

    def body(g_ref, nb_ref, p1_ref, w_ref, o_ref):
        g3 = _edge_rows(g_ref, nb_ref, p1_ref, w_ref, a, m, blk)

        @pl.when(pl.program_id(0) == 0)
        def _():
            o_ref[...] = jnp.zeros_like(o_ref)

        o_ref[0:1, :] += jnp.sum(g3, axis=(0, 1))[None, :]
        o_ref[1:2, :] += jnp.sum(g3 * g3, axis=(0, 1))[None, :]

    return pl.pallas_call(
        body,
        grid=(grid,),
        in_specs=[
            pl.BlockSpec((eblk, a), lambda i: (i, 0)),
            pl.BlockSpec((eblk, nbr2.shape[1]), lambda i: (i, 0)),
            pl.BlockSpec((blk, 2 * a), lambda i: (i, 0)),
            pl.BlockSpec(w.shape, lambda i: (0, 0)),
        ],
        out_specs=pl.BlockSpec((8, 2 * a), lambda i: (0, 0)),
        out_shape=jax.ShapeDtypeStruct((8, 2 * a), jnp.float32),
    )(g_rows, nbr2, p1, w)


def _gate_pass(g_rows, nbr2, p1, w, st1, gamma1, beta1, n, m, a):
    blk = _BLK
    eblk = blk * m
    grid = n // blk
    cnt = float(n * m)

    def body(g_ref, nb_ref, p1_ref, w_ref, st_ref, ga_ref, be_ref, s_ref, o2_ref):
        mean = st_ref[0:1, :] / cnt
        var = st_ref[1:2, :] / cnt - mean * mean
        scale = ga_ref[...] * jax.lax.rsqrt(var + _EPS)
        shift = be_ref[...] - mean * scale

        g3 = _edge_rows(g_ref, nb_ref, p1_ref, w_ref, a, m, blk)
        y = g3 * scale[None, :, :] + shift[None, :, :]
        filt = jax.nn.sigmoid(y[:, :, :a])
        core = jax.nn.softplus(y[:, :, a:])
        s_blk = jnp.sum(filt * core, axis=1)
        s_ref[...] = s_blk

        @pl.when(pl.program_id(0) == 0)
        def _():
            o2_ref[...] = jnp.zeros_like(o2_ref)

        o2_ref[0:1, :] += jnp.sum(s_blk, axis=0)[None, :]
        o2_ref[1:2, :] += jnp.sum(s_blk * s_blk, axis=0)[None, :]

    return pl.pallas_call(
        body,
        grid=(grid,),
        in_specs=[
            pl.BlockSpec((eblk, a), lambda i: (i, 0)),
            pl.BlockSpec((eblk, nbr2.shape[1]), lambda i: (i, 0)),
            pl.BlockSpec((blk, 2 * a), lambda i: (i, 0)),
            pl.BlockSpec(w.shape, lambda i: (0, 0)),
            pl.BlockSpec((8, 2 * a), lambda i: (0, 0)),
            pl.BlockSpec((1, 2 * a), lambda i: (0, 0)),
            pl.BlockSpec((1, 2 * a), lambda i: (0, 0)),
        ],
        out_specs=[
            pl.BlockSpec((blk, a), lambda i: (i, 0)),
            pl.BlockSpec((8, a), lambda i: (0, 0)),
        ],
        out_shape=[
            jax.ShapeDtypeStruct((n, a), jnp.float32),
            jax.ShapeDtypeStruct((8, a), jnp.float32),
        ],
    )(g_rows, nbr2, p1, w, st1, gamma1, beta1)


def _final_pass(x, s, st2, gamma2, beta2):
    n, a = x.shape
    blk = _BLK3
    cnt = float(n)

    def body(x_ref, s_ref, st_ref, ga_ref, be_ref, o_ref):
        mean = st_ref[0:1, :] / cnt
        var = st_ref[1:2, :] / cnt - mean * mean
        scale = ga_ref[...] * jax.lax.rsqrt(var + _EPS)
        shift = be_ref[...] - mean * scale
        o_ref[...] = jax.nn.softplus(x_ref[...] + s_ref[...] * scale + shift)

    return pl.pallas_call(
        body,
        grid=(n // blk,),
        in_specs=[
            pl.BlockSpec((blk, a), lambda i: (i, 0)),
            pl.BlockSpec((blk, a), lambda i: (i, 0)),
            pl.BlockSpec((8, a), lambda i: (0, 0)),
            pl.BlockSpec((1, a), lambda i: (0, 0)),
            pl.BlockSpec((1, a), lambda i: (0, 0)),
        ],
        out_specs=pl.BlockSpec((blk, a), lambda i: (i, 0)),
        out_shape=jax.ShapeDtypeStruct((n, a), jnp.float32),
    )(x, s, st2, gamma2, beta2)


def kernel(atom_in_fea, nbr_fea, nbr_fea_idx, W, b, gamma1, beta1, gamma2, beta2):
    n, m = nbr_fea_idx.shape
    a = atom_in_fea.shape[1]
    e = n * m

    chunk = _WIN * _NWORK
    epad = ((e + chunk - 1) // chunk) * chunk
    flat_idx = nbr_fea_idx.reshape(1, e)
    if epad != e:
        flat_idx = jnp.concatenate(
            [flat_idx, jnp.zeros((1, epad - e), jnp.int32)], axis=1
        )

    nbr2 = nbr_fea.reshape(e, nbr_fea.shape[2])
    b2d = b.reshape(1, 2 * a)
    g1 = gamma1.reshape(1, 2 * a)
    b1 = beta1.reshape(1, 2 * a)
    g2 = gamma2.reshape(1, a)
    b2 = beta2.reshape(1, a)

    p1, _ = _p1_kernel(atom_in_fea, W, b2d)
    g_rows = _sc_gather(atom_in_fea, flat_idx, epad)
    st1 = _stats_pass(g_rows, nbr2, p1, W, n, m, a)
    s, st2 = _gate_pass(g_rows, nbr2, p1, W, st1, g1, b1, n, m, a)
    return _final_pass(atom_in_fea, s, st2, gamma2=g2, beta2=b2)


# speedup vs baseline: 1.0421x; 1.0421x over previous
import functools

import jax
import jax.numpy as jnp
from jax.experimental import pallas as pl
from jax.experimental.pallas import tpu as pltpu
from jax.experimental.pallas import tpu_sc as plsc

_EPS = 1e-5
_WIN = 128
_NWORK = 32
_BLK = 200
_BLK3 = 1000


def _sc_gather(table, idx_pad, epad):
    a = table.shape[1]
    mesh = plsc.VectorSubcoreMesh(core_axis_name="core", subcore_axis_name="subcore")

    steps = epad // _WIN // _NWORK

    @functools.partial(
        pl.kernel,
        out_type=jax.ShapeDtypeStruct((epad, a), table.dtype),
        mesh=mesh,
    )
    def gather_kernel(x_hbm, i_hbm, o_hbm):
        def body(i_vmem, o_vmem):
            pltpu.sync_copy(x_hbm.at[i_vmem.at[0]], o_vmem)

        pltpu.emit_pipeline(
            body,
            grid=(_NWORK, steps),
            in_specs=[pl.BlockSpec((1, _WIN), lambda i, j: (0, i * steps + j))],
            out_specs=[pl.BlockSpec((_WIN, a), lambda i, j: (i * steps + j, 0))],
            core_axis_name=("core", "subcore"),
            dimension_semantics=(pltpu.PARALLEL, pltpu.ARBITRARY),
        )(i_hbm, o_hbm)

    return gather_kernel(table, idx_pad)


def _p1_kernel(x, w, b2d):
    n, a = x.shape
    c = w.shape[1]
    blk = _BLK3

    def body(x_ref, w_ref, b_ref, o_ref, xb_ref):
        w1 = w_ref[0:a, :]
        o_ref[...] = (
            jnp.dot(x_ref[...], w1, preferred_element_type=jnp.float32)
            + b_ref[...]
        )
        xb_ref[...] = x_ref[...].astype(jnp.bfloat16)

    return pl.pallas_call(
        body,
        grid=(n // blk,),
        in_specs=[
            pl.BlockSpec((blk, a), lambda i: (i, 0)),
            pl.BlockSpec(w.shape, lambda i: (0, 0)),
            pl.BlockSpec(b2d.shape, lambda i: (0, 0)),
        ],
        out_specs=[
            pl.BlockSpec((blk, c), lambda i: (i, 0)),
            pl.BlockSpec((blk, a), lambda i: (i, 0)),
        ],
        out_shape=[
            jax.ShapeDtypeStruct((n, c), jnp.float32),
            jax.ShapeDtypeStruct((n, a), jnp.bfloat16),
        ],
    )(x, w, b2d)


def _edge_rows(g_ref, nb_ref, p1_ref, w_ref, a, m, blk):
    w2 = w_ref[a : 2 * a, :].astype(jnp.bfloat16)
    w3 = w_ref[2 * a :, :].astype(jnp.bfloat16)
    g = jnp.dot(
        g_ref[...].astype(jnp.bfloat16), w2, preferred_element_type=jnp.float32
    )
    g = g + jnp.dot(
        nb_ref[...].astype(jnp.bfloat16), w3, preferred_element_type=jnp.float32
    )
    g3 = g.reshape(blk, m, 2 * a) + p1_ref[...][:, None, :]
    return g3


def _stats_pass(g_rows, nbr2, p1, w, n, m, a):
    blk = _BLK
    eblk = blk * m
    grid = n // blk

    def body(g_ref, nb_ref, p1_ref, w_ref, o_ref):
        g3 = _edge_rows(g_ref, nb_ref, p1_ref, w_ref, a, m, blk)

        @pl.when(pl.program_id(0) == 0)
        def _():
            o_ref[...] = jnp.zeros_like(o_ref)

        o_ref[0:1, :] += jnp.sum(g3, axis=(0, 1))[None, :]
        o_ref[1:2, :] += jnp.sum(g3 * g3, axis=(0, 1))[None, :]

    return pl.pallas_call(
        body,
        grid=(grid,),
        in_specs=[
            pl.BlockSpec((eblk, a), lambda i: (i, 0)),
            pl.BlockSpec((eblk, nbr2.shape[1]), lambda i: (i, 0)),
            pl.BlockSpec((blk, 2 * a), lambda i: (i, 0)),
            pl.BlockSpec(w.shape, lambda i: (0, 0)),
        ],
        out_specs=pl.BlockSpec((8, 2 * a), lambda i: (0, 0)),
        out_shape=jax.ShapeDtypeStruct((8, 2 * a), jnp.float32),
    )(g_rows, nbr2, p1, w)


def _gate_pass(g_rows, nbr2, p1, w, st1, gamma1, beta1, n, m, a):
    blk = _BLK
    eblk = blk * m
    grid = n // blk
    cnt = float(n * m)

    def body(g_ref, nb_ref, p1_ref, w_ref, st_ref, ga_ref, be_ref, s_ref, o2_ref):
        mean = st_ref[0:1, :] / cnt
        var = st_ref[1:2, :] / cnt - mean * mean
        scale = ga_ref[...] * jax.lax.rsqrt(var + _EPS)
        shift = be_ref[...] - mean * scale

        g3 = _edge_rows(g_ref, nb_ref, p1_ref, w_ref, a, m, blk)
        y = g3 * scale[None, :, :] + shift[None, :, :]
        filt = jax.nn.sigmoid(y[:, :, :a])
        core = jax.nn.softplus(y[:, :, a:])
        s_blk = jnp.sum(filt * core, axis=1)
        s_ref[...] = s_blk

        @pl.when(pl.program_id(0) == 0)
        def _():
            o2_ref[...] = jnp.zeros_like(o2_ref)

        o2_ref[0:1, :] += jnp.sum(s_blk, axis=0)[None, :]
        o2_ref[1:2, :] += jnp.sum(s_blk * s_blk, axis=0)[None, :]

    return pl.pallas_call(
        body,
        grid=(grid,),
        in_specs=[
            pl.BlockSpec((eblk, a), lambda i: (i, 0)),
            pl.BlockSpec((eblk, nbr2.shape[1]), lambda i: (i, 0)),
            pl.BlockSpec((blk, 2 * a), lambda i: (i, 0)),
            pl.BlockSpec(w.shape, lambda i: (0, 0)),
            pl.BlockSpec((8, 2 * a), lambda i: (0, 0)),
            pl.BlockSpec((1, 2 * a), lambda i: (0, 0)),
            pl.BlockSpec((1, 2 * a), lambda i: (0, 0)),
        ],
        out_specs=[
            pl.BlockSpec((blk, a), lambda i: (i, 0)),
            pl.BlockSpec((8, a), lambda i: (0, 0)),
        ],
        out_shape=[
            jax.ShapeDtypeStruct((n, a), jnp.float32),
            jax.ShapeDtypeStruct((8, a), jnp.float32),
        ],
    )(g_rows, nbr2, p1, w, st1, gamma1, beta1)


def _final_pass(x, s, st2, gamma2, beta2):
    n, a = x.shape
    blk = _BLK3
    cnt = float(n)

    def body(x_ref, s_ref, st_ref, ga_ref, be_ref, o_ref):
        mean = st_ref[0:1, :] / cnt
        var = st_ref[1:2, :] / cnt - mean * mean
        scale = ga_ref[...] * jax.lax.rsqrt(var + _EPS)
        shift = be_ref[...] - mean * scale
        o_ref[...] = jax.nn.softplus(x_ref[...] + s_ref[...] * scale + shift)

    return pl.pallas_call(
        body,
        grid=(n // blk,),
        in_specs=[
            pl.BlockSpec((blk, a), lambda i: (i, 0)),
            pl.BlockSpec((blk, a), lambda i: (i, 0)),
            pl.BlockSpec((8, a), lambda i: (0, 0)),
            pl.BlockSpec((1, a), lambda i: (0, 0)),
            pl.BlockSpec((1, a), lambda i: (0, 0)),
        ],
        out_specs=pl.BlockSpec((blk, a), lambda i: (i, 0)),
        out_shape=jax.ShapeDtypeStruct((n, a), jnp.float32),
    )(x, s, st2, gamma2, beta2)


def kernel(atom_in_fea, nbr_fea, nbr_fea_idx, W, b, gamma1, beta1, gamma2, beta2):
    n, m = nbr_fea_idx.shape
    a = atom_in_fea.shape[1]
    e = n * m

    chunk = _WIN * _NWORK
    epad = ((e + chunk - 1) // chunk) * chunk
    flat_idx = nbr_fea_idx.reshape(1, e)
    if epad != e:
        flat_idx = jnp.concatenate(
            [flat_idx, jnp.zeros((1, epad - e), jnp.int32)], axis=1
        )

    nbr2 = nbr_fea.reshape(e, nbr_fea.shape[2])
    b2d = b.reshape(1, 2 * a)
    g1 = gamma1.reshape(1, 2 * a)
    b1 = beta1.reshape(1, 2 * a)
    g2 = gamma2.reshape(1, a)
    b2 = beta2.reshape(1, a)

    p1, _ = _p1_kernel(atom_in_fea, W, b2d)
    g_rows = _sc_gather(atom_in_fea, flat_idx, epad)
    st1 = _stats_pass(g_rows, nbr2, p1, W, n, m, a)
    s, st2 = _gate_pass(g_rows, nbr2, p1, W, st1, g1, b1, n, m, a)
    return _final_pass(atom_in_fea, s, st2, gamma2=g2, beta2=b2)
